# Initial kernel scaffold; baseline (speedup 1.0000x reference)
import jax
import jax.numpy as jnp
from jax.experimental import pallas as pl


def kernel(x, W_enc, b_enc, W_dec, b_dec):
    raise NotImplementedError("write your pallas kernel here")


# speedup vs baseline: 27.5713x; 27.5713x over previous
import functools

import jax
import jax.numpy as jnp
from jax import lax
from jax.experimental import pallas as pl
from jax.experimental.pallas import tpu as pltpu
from jax.experimental.pallas import tpu_sc as plsc
from jax.scipy.special import ndtri

D_IN = 1280
NUM_LATENTS = 20480
K_TOP = 32
N_TOK = 2048
NL = N_TOK * NUM_LATENTS
TOTAL_K = K_TOP * N_TOK

NW = 32
TOK_PER_W = N_TOK // NW
CAP_T = 12288
ROWS = NUM_LATENTS // 128

TOK_BLK = 256
LAT_BLK = 2048


def _flatten_kernel(p_ref, out_ref, stats_ref):
    blk = p_ref[...]
    out_ref[...] = blk.reshape(TOK_BLK, LAT_BLK // 128, 128)
    pos = (blk > 0.0).astype(jnp.float32)
    s = jnp.stack([
        jnp.sum(pos),
        jnp.sum(blk),
        jnp.sum(blk * blk),
        jnp.max(blk),
        0.0, 0.0, 0.0, 0.0,
    ])
    stats_ref[...] = s.reshape(1, 1, 8)


def _flatten(pre2d):
    gi, gj = N_TOK // TOK_BLK, NUM_LATENTS // LAT_BLK
    return pl.pallas_call(
        _flatten_kernel,
        grid=(gi, gj),
        in_specs=[pl.BlockSpec((TOK_BLK, LAT_BLK), lambda i, j: (i, j))],
        out_specs=[
            pl.BlockSpec((TOK_BLK, LAT_BLK // 128, 128),
                         lambda i, j: (i, j, 0)),
            pl.BlockSpec((1, 1, 8), lambda i, j, _gj=gj: (i * _gj + j, 0, 0)),
        ],
        out_shape=[
            jax.ShapeDtypeStruct((N_TOK, ROWS, 128), jnp.float32),
            jax.ShapeDtypeStruct((gi * gj, 1, 8), jnp.float32),
        ],
    )(pre2d)


def _compact_kernel(flat_hbm, t0_hbm, vals_hbm, idx_hbm, cnt_hbm,
                    buf0, buf1, t0_v, vals_v, idx_v, cnt_v, sem0, sem1, semt):
    nc = 2
    wid = lax.axis_index("s") * nc + lax.axis_index("c")
    tok0 = wid * TOK_PER_W

    pltpu.async_copy(t0_hbm, t0_v, semt).wait()
    lanes = lax.iota(jnp.int32, 16)
    onesv = jnp.ones((16,), jnp.int32)
    zerosv = jnp.zeros((16,), jnp.int32)
    capv = jnp.full((16,), CAP_T, jnp.int32)

    def process(buf, tok, cursor):
        base = lax.broadcast(tok * NUM_LATENTS, (16,))

        def body(i, cur):
            r = i // 8
            c = i % 8
            off = lax.broadcast(r * 128 + c * 16, (16,))
            v = buf[r, pl.ds(c * 16, 16)]
            m = v > t0_v[...]
            m32 = jnp.where(m, onesv, zerosv)
            cum = plsc.cumsum(m32)
            pos = (cur + cum) - onesv
            wm = m & (pos < capv)
            iv = (base + off) + lanes
            plsc.store_scatter(vals_v, [pos], v, mask=wm)
            plsc.store_scatter(idx_v, [pos], iv, mask=wm)
            return cur + plsc.all_reduce_population_count(m)

        return lax.fori_loop(0, ROWS * 8, body, cursor)

    cursor = jnp.zeros((16,), jnp.int32)
    pltpu.async_copy(flat_hbm.at[tok0], buf0, sem0)
    npair = TOK_PER_W // 2

    def pair_body(p, cursor):
        tok_a = tok0 + 2 * p
        pltpu.make_async_copy(flat_hbm.at[tok_a], buf0, sem0).wait()
        pltpu.async_copy(flat_hbm.at[tok_a + 1], buf1, sem1)
        cursor = process(buf0, tok_a, cursor)
        pltpu.make_async_copy(flat_hbm.at[tok_a + 1], buf1, sem1).wait()

        @pl.when(p + 1 < npair)
        def _():
            pltpu.async_copy(flat_hbm.at[tok_a + 2], buf0, sem0)

        cursor = process(buf1, tok_a + 1, cursor)
        return cursor

    cursor = lax.fori_loop(0, npair, pair_body, cursor)
    cnt_v[...] = cursor
    pltpu.async_copy(cnt_v, cnt_hbm.at[wid], semt).wait()
    pltpu.async_copy(vals_v, vals_hbm.at[wid], sem0).wait()
    pltpu.async_copy(idx_v, idx_hbm.at[wid], sem1).wait()


def _compact(flat3d, t0_arr):
    mesh = plsc.VectorSubcoreMesh(core_axis_name="c", subcore_axis_name="s")
    kern = pl.kernel(
        _compact_kernel,
        mesh=mesh,
        compiler_params=pltpu.CompilerParams(needs_layout_passes=False),
        out_type=[
            jax.ShapeDtypeStruct((NW, CAP_T), jnp.float32),
            jax.ShapeDtypeStruct((NW, CAP_T), jnp.int32),
            jax.ShapeDtypeStruct((NW, 16), jnp.int32),
        ],
        scratch_types=[
            pltpu.VMEM((ROWS, 128), jnp.float32),
            pltpu.VMEM((ROWS, 128), jnp.float32),
            pltpu.VMEM((16,), jnp.float32),
            pltpu.VMEM((CAP_T,), jnp.float32),
            pltpu.VMEM((CAP_T,), jnp.int32),
            pltpu.VMEM((16,), jnp.int32),
            pltpu.SemaphoreType.DMA,
            pltpu.SemaphoreType.DMA,
            pltpu.SemaphoreType.DMA,
        ],
    )
    return kern(flat3d, t0_arr)


def _f32_bits(x):
    return lax.bitcast_convert_type(x.astype(jnp.float32), jnp.int32)


def _bits_f32(b):
    return lax.bitcast_convert_type(b, jnp.float32)


def _select_candidates(flat3d, stats):
    st = stats.reshape(-1, 8)
    sumsq = jnp.sum(st[:, 2])
    vmax = jnp.max(st[:, 3])
    sigma = jnp.sqrt(2.0 * sumsq / NL + 1e-30)
    q = 2.0 * TOTAL_K / NL
    t_est = sigma * ndtri(1.0 - q).astype(jnp.float32)
    t_est = jnp.clip(t_est, 1e-30, jnp.maximum(vmax * 0.999, 1e-30))

    def run(t0):
        t0v = jnp.broadcast_to(t0.astype(jnp.float32), (16,))
        return _compact(flat3d, t0v)

    vals, idx, cnt = run(t_est)

    lo0 = jnp.int32(0)
    hi0 = _f32_bits(jnp.maximum(vmax, jnp.float32(1e-30))) + 1
    state = (lo0, hi0, _f32_bits(t_est), vals, idx, cnt, jnp.int32(0))

    def cond(s):
        lo, hi, tb, vals_, idx_, cnt_, it = s
        c = cnt_[:, 0]
        bad = (jnp.sum(c) < TOTAL_K) | (jnp.max(c) > CAP_T)
        return bad & (it < 40)

    def body(s):
        lo, hi, tb, vals_, idx_, cnt_, it = s
        c = cnt_[:, 0]
        too_high = jnp.sum(c) < TOTAL_K
        lo2 = jnp.where(too_high, lo, tb)
        hi2 = jnp.where(too_high, tb, hi)
        tb2 = (lo2 + hi2) // 2
        vals2, idx2, cnt2 = run(_bits_f32(tb2))
        return (lo2, hi2, tb2, vals2, idx2, cnt2, it + 1)

    _, _, _, vals, idx, cnt, _ = lax.while_loop(cond, body, state)
    return vals, idx, cnt[:, 0]


def kernel(x, W_enc, b_enc, W_dec, b_dec):
    B, S, E = x.shape
    xf = x.reshape(B * S, E)

    pre_acts = jax.nn.relu((xf - b_dec) @ W_enc.T + b_enc)

    flat3d, stats = _flatten(pre_acts)
    cand_vals, cand_idx, cand_cnt = _select_candidates(flat3d, stats)

    slot = jnp.arange(CAP_T, dtype=jnp.int32)[None, :]
    valid = slot < cand_cnt[:, None]
    v_flat = jnp.where(valid, cand_vals, -1.0).reshape(-1)
    i_flat = jnp.where(valid, cand_idx, jnp.int32(2**31 - 1)).reshape(-1)

    key1 = -v_flat
    _, top_idx, top_vals = lax.sort((key1, i_flat, v_flat), num_keys=2)
    top_vals = top_vals[:TOTAL_K]
    top_idx = top_idx[:TOTAL_K]

    l0_loss = jnp.sum(stats.reshape(-1, 8)[:, 0]) / N_TOK

    flat_dense = jnp.zeros((NL,), jnp.float32).at[top_idx].set(top_vals)
    top_acts = flat_dense.reshape(N_TOK, NUM_LATENTS)
    top_indices = (top_idx % NUM_LATENTS).reshape(N_TOK, K_TOP)

    sae_out = top_acts @ W_dec + b_dec
    e = (sae_out - xf).astype(jnp.float32)
    total_variance = jnp.sum(((xf - xf.mean(0)).astype(jnp.float32)) ** 2)
    auxk_loss = jnp.float32(0.0)
    l2_loss = jnp.sum(e ** 2)
    fvu = l2_loss / total_variance
    per_token_l2_loss = jnp.squeeze(jnp.sum((sae_out - xf) ** 2, axis=-1))
    per_token_total_variance = jnp.sum((xf - xf.mean(0)) ** 2, axis=-1)
    explained_variance = 1.0 - per_token_l2_loss / per_token_total_variance
    return (sae_out, top_acts, top_indices, fvu, l0_loss, l2_loss,
            auxk_loss, explained_variance)


# speedup vs baseline: 32.6700x; 1.1849x over previous
import functools

import jax
import jax.numpy as jnp
from jax import lax
from jax.experimental import pallas as pl
from jax.experimental.pallas import tpu as pltpu
from jax.experimental.pallas import tpu_sc as plsc
from jax.scipy.special import ndtri

D_IN = 1280
NUM_LATENTS = 20480
K_TOP = 32
N_TOK = 2048
NL = N_TOK * NUM_LATENTS
TOTAL_K = K_TOP * N_TOK

NW = 32
TOK_PER_W = N_TOK // NW
CAP_T = 4096
ROWS = NUM_LATENTS // 128

TOK_BLK = 256
LAT_BLK = 2048
GI = N_TOK // TOK_BLK
GK = NUM_LATENTS // LAT_BLK


def _flatten_kernel(p_ref, out_ref, stats_ref):
    blk = p_ref[...]
    out_ref[...] = blk.reshape(TOK_BLK, LAT_BLK // 128, 128)
    pos = (blk > 0.0).astype(jnp.float32)
    s = jnp.stack([
        jnp.sum(pos),
        jnp.sum(blk),
        jnp.sum(blk * blk),
        jnp.max(blk),
        0.0, 0.0, 0.0, 0.0,
    ])
    stats_ref[...] = s.reshape(1, 1, 8)


def _flatten(pre2d):
    return pl.pallas_call(
        _flatten_kernel,
        grid=(GI, GK),
        in_specs=[pl.BlockSpec((TOK_BLK, LAT_BLK), lambda i, j: (i, j))],
        out_specs=[
            pl.BlockSpec((TOK_BLK, LAT_BLK // 128, 128),
                         lambda i, j: (i, j, 0)),
            pl.BlockSpec((1, 1, 8), lambda i, j: (i * GK + j, 0, 0)),
        ],
        out_shape=[
            jax.ShapeDtypeStruct((N_TOK, ROWS, 128), jnp.float32),
            jax.ShapeDtypeStruct((GI * GK, 1, 8), jnp.float32),
        ],
    )(pre2d)


def _compact_kernel(flat_hbm, t0_hbm, vals_hbm, idx_hbm, cnt_hbm,
                    buf0, buf1, t0_v, vals_v, idx_v, cnt_v, sem0, sem1, semt):
    nc = 2
    wid = lax.axis_index("s") * nc + lax.axis_index("c")
    tok0 = wid * TOK_PER_W

    pltpu.async_copy(t0_hbm, t0_v, semt).wait()
    t0x = t0_v[...]
    lanes = lax.iota(jnp.int32, 16)
    onesv = jnp.ones((16,), jnp.int32)
    zerosv = jnp.zeros((16,), jnp.int32)
    capv = jnp.full((16,), CAP_T, jnp.int32)
    coffs = [jnp.full((16,), c * 16, jnp.int32) + lanes for c in range(8)]

    def process(buf, tok, cursor):
        base = lax.broadcast(tok * NUM_LATENTS, (16,))

        def body(r, cur):
            rb = base + lax.broadcast(r * 128, (16,))
            for c in range(8):
                v = buf[r, pl.ds(c * 16, 16)]
                m = v > t0x
                m32 = jnp.where(m, onesv, zerosv)
                cum = plsc.cumsum(m32)
                pos = (cur + cum) - onesv
                wm = m & (pos < capv)
                iv = rb + coffs[c]
                plsc.store_scatter(vals_v, [pos], v, mask=wm)
                plsc.store_scatter(idx_v, [pos], iv, mask=wm)
                cur = cur + plsc.all_reduce_population_count(m)
            return cur

        return lax.fori_loop(0, ROWS, body, cursor)

    cursor = jnp.zeros((16,), jnp.int32)
    pltpu.async_copy(flat_hbm.at[tok0], buf0, sem0)
    npair = TOK_PER_W // 2

    def pair_body(p, cursor):
        tok_a = tok0 + 2 * p
        pltpu.make_async_copy(flat_hbm.at[tok_a], buf0, sem0).wait()
        pltpu.async_copy(flat_hbm.at[tok_a + 1], buf1, sem1)
        cursor = process(buf0, tok_a, cursor)
        pltpu.make_async_copy(flat_hbm.at[tok_a + 1], buf1, sem1).wait()

        @pl.when(p + 1 < npair)
        def _():
            pltpu.async_copy(flat_hbm.at[tok_a + 2], buf0, sem0)

        cursor = process(buf1, tok_a + 1, cursor)
        return cursor

    cursor = lax.fori_loop(0, npair, pair_body, cursor)
    cnt_v[...] = cursor
    pltpu.async_copy(cnt_v, cnt_hbm.at[wid], semt).wait()
    pltpu.async_copy(vals_v, vals_hbm.at[wid], sem0).wait()
    pltpu.async_copy(idx_v, idx_hbm.at[wid], sem1).wait()


def _compact(flat3d, t0_arr):
    mesh = plsc.VectorSubcoreMesh(core_axis_name="c", subcore_axis_name="s")
    kern = pl.kernel(
        _compact_kernel,
        mesh=mesh,
        compiler_params=pltpu.CompilerParams(needs_layout_passes=False),
        out_type=[
            jax.ShapeDtypeStruct((NW, CAP_T), jnp.float32),
            jax.ShapeDtypeStruct((NW, CAP_T), jnp.int32),
            jax.ShapeDtypeStruct((NW, 16), jnp.int32),
        ],
        scratch_types=[
            pltpu.VMEM((ROWS, 128), jnp.float32),
            pltpu.VMEM((ROWS, 128), jnp.float32),
            pltpu.VMEM((16,), jnp.float32),
            pltpu.VMEM((CAP_T,), jnp.float32),
            pltpu.VMEM((CAP_T,), jnp.int32),
            pltpu.VMEM((16,), jnp.int32),
            pltpu.SemaphoreType.DMA,
            pltpu.SemaphoreType.DMA,
            pltpu.SemaphoreType.DMA,
        ],
    )
    return kern(flat3d, t0_arr)


def _f32_bits(x):
    return lax.bitcast_convert_type(x.astype(jnp.float32), jnp.int32)


def _bits_f32(b):
    return lax.bitcast_convert_type(b, jnp.float32)


def _select_candidates(flat3d, stats):
    st = stats.reshape(-1, 8)
    sumsq = jnp.sum(st[:, 2])
    vmax = jnp.max(st[:, 3])
    sigma = jnp.sqrt(2.0 * sumsq / NL + 1e-30)
    q = 1.45 * TOTAL_K / NL
    t_est = sigma * ndtri(1.0 - q).astype(jnp.float32)
    t_est = jnp.clip(t_est, 1e-30, jnp.maximum(vmax * 0.999, 1e-30))

    def run(t0):
        t0v = jnp.broadcast_to(t0.astype(jnp.float32), (16,))
        return _compact(flat3d, t0v)

    vals, idx, cnt = run(t_est)

    lo0 = jnp.int32(0)
    hi0 = _f32_bits(jnp.maximum(vmax, jnp.float32(1e-30))) + 1
    state = (lo0, hi0, _f32_bits(t_est), vals, idx, cnt, jnp.int32(0))

    def cond(s):
        lo, hi, tb, vals_, idx_, cnt_, it = s
        c = cnt_[:, 0]
        bad = (jnp.sum(c) < TOTAL_K) | (jnp.max(c) > CAP_T)
        return bad & (it < 40)

    def body(s):
        lo, hi, tb, vals_, idx_, cnt_, it = s
        c = cnt_[:, 0]
        too_high = jnp.sum(c) < TOTAL_K
        lo2 = jnp.where(too_high, lo, tb)
        hi2 = jnp.where(too_high, tb, hi)
        tb2 = (lo2 + hi2) // 2
        vals2, idx2, cnt2 = run(_bits_f32(tb2))
        return (lo2, hi2, tb2, vals2, idx2, cnt2, it + 1)

    _, _, _, vals, idx, cnt, _ = lax.while_loop(cond, body, state)
    return vals, idx, cnt[:, 0]


def _stage_pass(v, x, s, d, n_arr):
    if d >= 128:
        axis, shift = 0, d // 128
    else:
        axis, shift = 1, d
    is_a = (n_arr & d) == 0
    asc = (n_arr & (1 << s)) == 0
    pv = jnp.where(is_a, jnp.roll(v, -shift, axis=axis),
                   jnp.roll(v, shift, axis=axis))
    px = jnp.where(is_a, jnp.roll(x, -shift, axis=axis),
                   jnp.roll(x, shift, axis=axis))
    a_v = jnp.where(is_a, v, pv)
    b_v = jnp.where(is_a, pv, v)
    a_x = jnp.where(is_a, x, px)
    b_x = jnp.where(is_a, px, x)
    af = (a_v > b_v) | ((a_v == b_v) & (a_x < b_x))
    fv = jnp.where(af, a_v, b_v)
    fx = jnp.where(af, a_x, b_x)
    sv = jnp.where(af, b_v, a_v)
    sx = jnp.where(af, b_x, a_x)
    keep_first = is_a == asc
    return jnp.where(keep_first, fv, sv), jnp.where(keep_first, fx, sx)


def _bitonic_kernel(vin_ref, iin_ref, vout_ref, iout_ref, NLOG=0):
    v = vin_ref[...]
    x = iin_ref[...]
    R = v.shape[0]
    ri = lax.broadcasted_iota(jnp.int32, (R, 128), 0)
    ci = lax.broadcasted_iota(jnp.int32, (R, 128), 1)
    n_arr = ri * 128 + ci
    for s in range(1, NLOG + 1):
        for dp in range(s - 1, -1, -1):
            v, x = _stage_pass(v, x, s, 1 << dp, n_arr)
    vout_ref[...] = v
    iout_ref[...] = x


def _bitonic_sort(vals, idx):
    N = vals.shape[0]
    R = N // 128
    NLOG = N.bit_length() - 1
    v2 = vals.reshape(R, 128)
    i2 = idx.reshape(R, 128)
    vo, io = pl.pallas_call(
        functools.partial(_bitonic_kernel, NLOG=NLOG),
        in_specs=[pl.BlockSpec((R, 128), lambda: (0, 0)),
                  pl.BlockSpec((R, 128), lambda: (0, 0))],
        out_specs=[pl.BlockSpec((R, 128), lambda: (0, 0)),
                   pl.BlockSpec((R, 128), lambda: (0, 0))],
        out_shape=[jax.ShapeDtypeStruct((R, 128), jnp.float32),
                   jax.ShapeDtypeStruct((R, 128), jnp.int32)],
    )(v2, i2)
    return vo.reshape(N), io.reshape(N)


def _mean_kernel(x_ref, out_ref):
    @pl.when(pl.program_id(0) == 0)
    def _():
        out_ref[...] = jnp.zeros_like(out_ref)

    out_ref[...] += jnp.sum(x_ref[...], axis=0, keepdims=True)


def _col_mean(xf):
    s = pl.pallas_call(
        _mean_kernel,
        grid=(GI,),
        in_specs=[pl.BlockSpec((TOK_BLK, D_IN), lambda i: (i, 0))],
        out_specs=pl.BlockSpec((1, D_IN), lambda i: (0, 0)),
        out_shape=jax.ShapeDtypeStruct((1, D_IN), jnp.float32),
    )(xf)
    return s / N_TOK


def _decode_kernel(ta_ref, wd_ref, xf_ref, xbar_ref, bdec_ref,
                   sae_ref, ptl2_ref, pttv_ref, l2_ref, tv_ref, acc):
    k = pl.program_id(1)

    @pl.when(k == 0)
    def _():
        acc[...] = jnp.zeros_like(acc)

    acc[...] += jnp.dot(ta_ref[...], wd_ref[...],
                        preferred_element_type=jnp.float32)

    @pl.when(k == GK - 1)
    def _():
        i = pl.program_id(0)
        sae = acc[...] + bdec_ref[...]
        sae_ref[...] = sae
        xfb = xf_ref[...]
        e = sae - xfb
        ptl2 = jnp.sum(e * e, axis=1)
        d = xfb - xbar_ref[...]
        pttv = jnp.sum(d * d, axis=1)
        ptl2_ref[...] = ptl2.reshape(1, 1, TOK_BLK)
        pttv_ref[...] = pttv.reshape(1, 1, TOK_BLK)

        @pl.when(i == 0)
        def _():
            l2_ref[...] = jnp.zeros_like(l2_ref)
            tv_ref[...] = jnp.zeros_like(tv_ref)

        l2_ref[...] += jnp.sum(ptl2).reshape(1, 1)
        tv_ref[...] += jnp.sum(pttv).reshape(1, 1)


def _decode_stats(top_acts, W_dec, xf, xbar, b_dec):
    return pl.pallas_call(
        _decode_kernel,
        grid=(GI, GK),
        in_specs=[
            pl.BlockSpec((TOK_BLK, LAT_BLK), lambda i, k: (i, k)),
            pl.BlockSpec((LAT_BLK, D_IN), lambda i, k: (k, 0)),
            pl.BlockSpec((TOK_BLK, D_IN), lambda i, k: (i, 0)),
            pl.BlockSpec((1, D_IN), lambda i, k: (0, 0)),
            pl.BlockSpec((1, D_IN), lambda i, k: (0, 0)),
        ],
        out_specs=[
            pl.BlockSpec((TOK_BLK, D_IN), lambda i, k: (i, 0)),
            pl.BlockSpec((1, 1, TOK_BLK), lambda i, k: (i, 0, 0)),
            pl.BlockSpec((1, 1, TOK_BLK), lambda i, k: (i, 0, 0)),
            pl.BlockSpec((1, 1), lambda i, k: (0, 0)),
            pl.BlockSpec((1, 1), lambda i, k: (0, 0)),
        ],
        out_shape=[
            jax.ShapeDtypeStruct((N_TOK, D_IN), jnp.float32),
            jax.ShapeDtypeStruct((GI, 1, TOK_BLK), jnp.float32),
            jax.ShapeDtypeStruct((GI, 1, TOK_BLK), jnp.float32),
            jax.ShapeDtypeStruct((1, 1), jnp.float32),
            jax.ShapeDtypeStruct((1, 1), jnp.float32),
        ],
        scratch_shapes=[pltpu.VMEM((TOK_BLK, D_IN), jnp.float32)],
    )(top_acts, W_dec, xf, xbar, b_dec)


def kernel(x, W_enc, b_enc, W_dec, b_dec):
    B, S, E = x.shape
    xf = x.reshape(B * S, E)

    pre_acts = jax.nn.relu((xf - b_dec) @ W_enc.T + b_enc)

    flat3d, stats = _flatten(pre_acts)
    cand_vals, cand_idx, cand_cnt = _select_candidates(flat3d, stats)

    slot = jnp.arange(CAP_T, dtype=jnp.int32)[None, :]
    valid = slot < cand_cnt[:, None]
    v_flat = jnp.where(valid, cand_vals, -1.0).reshape(-1)
    i_flat = jnp.where(valid, cand_idx, jnp.int32(2**31 - 1)).reshape(-1)

    sv, si = _bitonic_sort(v_flat, i_flat)
    top_vals = sv[:TOTAL_K]
    top_idx = si[:TOTAL_K]

    l0_loss = jnp.sum(stats.reshape(-1, 8)[:, 0]) / N_TOK

    flat_dense = jnp.zeros((NL,), jnp.float32).at[top_idx].set(top_vals)
    top_acts = flat_dense.reshape(N_TOK, NUM_LATENTS)
    top_indices = (top_idx % NUM_LATENTS).reshape(N_TOK, K_TOP)

    xbar = _col_mean(xf)
    sae_out, ptl2_3d, pttv_3d, l2s, tvs = _decode_stats(
        top_acts, W_dec, xf, xbar, b_dec.reshape(1, D_IN))

    per_token_l2_loss = ptl2_3d.reshape(N_TOK)
    per_token_total_variance = pttv_3d.reshape(N_TOK)
    l2_loss = l2s.reshape(())
    total_variance = tvs.reshape(())
    auxk_loss = jnp.float32(0.0)
    fvu = l2_loss / total_variance
    explained_variance = 1.0 - per_token_l2_loss / per_token_total_variance
    return (sae_out, top_acts, top_indices, fvu, l0_loss, l2_loss,
            auxk_loss, explained_variance)


# speedup vs baseline: 53.8952x; 1.6497x over previous
import functools

import jax
import jax.numpy as jnp
from jax import lax
from jax.experimental import pallas as pl
from jax.experimental.pallas import tpu as pltpu
from jax.experimental.pallas import tpu_sc as plsc
from jax.scipy.special import ndtri

D_IN = 1280
NUM_LATENTS = 20480
K_TOP = 32
N_TOK = 2048
NL = N_TOK * NUM_LATENTS
TOTAL_K = K_TOP * N_TOK

NW = 32
TOK_PER_W = N_TOK // NW
CAP_T = 4096
ROWS = NUM_LATENTS // 128

TOK_BLK = 256
LAT_BLK = 2048
GI = N_TOK // TOK_BLK
GK = NUM_LATENTS // LAT_BLK


def _flatten_kernel(p_ref, out_ref, stats_ref):
    blk = p_ref[...]
    out_ref[...] = blk.reshape(TOK_BLK, LAT_BLK // 128, 128)
    pos = (blk > 0.0).astype(jnp.float32)
    s = jnp.stack([
        jnp.sum(pos),
        jnp.sum(blk),
        jnp.sum(blk * blk),
        jnp.max(blk),
        0.0, 0.0, 0.0, 0.0,
    ])
    stats_ref[...] = s.reshape(1, 1, 8)


def _flatten(pre2d):
    return pl.pallas_call(
        _flatten_kernel,
        grid=(GI, GK),
        in_specs=[pl.BlockSpec((TOK_BLK, LAT_BLK), lambda i, j: (i, j))],
        out_specs=[
            pl.BlockSpec((TOK_BLK, LAT_BLK // 128, 128),
                         lambda i, j: (i, j, 0)),
            pl.BlockSpec((1, 1, 8), lambda i, j: (i * GK + j, 0, 0)),
        ],
        out_shape=[
            jax.ShapeDtypeStruct((N_TOK, ROWS, 128), jnp.float32),
            jax.ShapeDtypeStruct((GI * GK, 1, 8), jnp.float32),
        ],
    )(pre2d)


def _compact_kernel(flat_hbm, t0_hbm, vals_hbm, idx_hbm, cnt_hbm,
                    buf0, buf1, t0_v, vals_v, idx_v, cnt_v, sem0, sem1, semt):
    nc = 2
    wid = lax.axis_index("s") * nc + lax.axis_index("c")
    tok0 = wid * TOK_PER_W

    pltpu.async_copy(t0_hbm, t0_v, semt).wait()
    t0x = t0_v[...]
    lanes = lax.iota(jnp.int32, 16)
    onesv = jnp.ones((16,), jnp.int32)
    zerosv = jnp.zeros((16,), jnp.int32)
    capv = jnp.full((16,), CAP_T, jnp.int32)
    coffs = [jnp.full((16,), c * 16, jnp.int32) + lanes for c in range(8)]

    def process(buf, tok, cursor):
        base = lax.broadcast(tok * NUM_LATENTS, (16,))

        def body(r, cur):
            rb = base + lax.broadcast(r * 128, (16,))
            vs = [buf[r, pl.ds(c * 16, 16)] for c in range(8)]
            ms = [v > t0x for v in vs]
            pcs = [plsc.all_reduce_population_count(m) for m in ms]
            cums = [plsc.cumsum(jnp.where(m, onesv, zerosv)) for m in ms]
            offs = [cur]
            for c in range(7):
                offs.append(offs[c] + pcs[c])
            for c in range(8):
                pos = (offs[c] + cums[c]) - onesv
                wm = ms[c] & (pos < capv)
                iv = rb + coffs[c]
                plsc.store_scatter(vals_v, [pos], vs[c], mask=wm)
                plsc.store_scatter(idx_v, [pos], iv, mask=wm)
            return offs[7] + pcs[7]

        return lax.fori_loop(0, ROWS, body, cursor)

    cursor = jnp.zeros((16,), jnp.int32)
    pltpu.async_copy(flat_hbm.at[tok0], buf0, sem0)
    npair = TOK_PER_W // 2

    def pair_body(p, cursor):
        tok_a = tok0 + 2 * p
        pltpu.make_async_copy(flat_hbm.at[tok_a], buf0, sem0).wait()
        pltpu.async_copy(flat_hbm.at[tok_a + 1], buf1, sem1)
        cursor = process(buf0, tok_a, cursor)
        pltpu.make_async_copy(flat_hbm.at[tok_a + 1], buf1, sem1).wait()

        @pl.when(p + 1 < npair)
        def _():
            pltpu.async_copy(flat_hbm.at[tok_a + 2], buf0, sem0)

        cursor = process(buf1, tok_a + 1, cursor)
        return cursor

    cursor = lax.fori_loop(0, npair, pair_body, cursor)
    cnt_v[...] = cursor
    pltpu.async_copy(cnt_v, cnt_hbm.at[wid], semt).wait()
    pltpu.async_copy(vals_v, vals_hbm.at[wid], sem0).wait()
    pltpu.async_copy(idx_v, idx_hbm.at[wid], sem1).wait()


def _compact(flat3d, t0_arr):
    mesh = plsc.VectorSubcoreMesh(core_axis_name="c", subcore_axis_name="s")
    kern = pl.kernel(
        _compact_kernel,
        mesh=mesh,
        compiler_params=pltpu.CompilerParams(needs_layout_passes=False),
        out_type=[
            jax.ShapeDtypeStruct((NW, CAP_T), jnp.float32),
            jax.ShapeDtypeStruct((NW, CAP_T), jnp.int32),
            jax.ShapeDtypeStruct((NW, 16), jnp.int32),
        ],
        scratch_types=[
            pltpu.VMEM((ROWS, 128), jnp.float32),
            pltpu.VMEM((ROWS, 128), jnp.float32),
            pltpu.VMEM((16,), jnp.float32),
            pltpu.VMEM((CAP_T,), jnp.float32),
            pltpu.VMEM((CAP_T,), jnp.int32),
            pltpu.VMEM((16,), jnp.int32),
            pltpu.SemaphoreType.DMA,
            pltpu.SemaphoreType.DMA,
            pltpu.SemaphoreType.DMA,
        ],
    )
    return kern(flat3d, t0_arr)


def _f32_bits(x):
    return lax.bitcast_convert_type(x.astype(jnp.float32), jnp.int32)


def _bits_f32(b):
    return lax.bitcast_convert_type(b, jnp.float32)


def _select_candidates(flat3d, stats):
    st = stats.reshape(-1, 8)
    sumsq = jnp.sum(st[:, 2])
    vmax = jnp.max(st[:, 3])
    sigma = jnp.sqrt(2.0 * sumsq / NL + 1e-30)
    q = 1.45 * TOTAL_K / NL
    t_est = sigma * ndtri(1.0 - q).astype(jnp.float32)
    t_est = jnp.clip(t_est, 1e-30, jnp.maximum(vmax * 0.999, 1e-30))

    def run(t0):
        t0v = jnp.broadcast_to(t0.astype(jnp.float32), (16,))
        return _compact(flat3d, t0v)

    vals, idx, cnt = run(t_est)

    lo0 = jnp.int32(0)
    hi0 = _f32_bits(jnp.maximum(vmax, jnp.float32(1e-30))) + 1
    state = (lo0, hi0, _f32_bits(t_est), vals, idx, cnt, jnp.int32(0))

    def cond(s):
        lo, hi, tb, vals_, idx_, cnt_, it = s
        c = cnt_[:, 0]
        bad = (jnp.sum(c) < TOTAL_K) | (jnp.max(c) > CAP_T)
        return bad & (it < 40)

    def body(s):
        lo, hi, tb, vals_, idx_, cnt_, it = s
        c = cnt_[:, 0]
        too_high = jnp.sum(c) < TOTAL_K
        lo2 = jnp.where(too_high, lo, tb)
        hi2 = jnp.where(too_high, tb, hi)
        tb2 = (lo2 + hi2) // 2
        vals2, idx2, cnt2 = run(_bits_f32(tb2))
        return (lo2, hi2, tb2, vals2, idx2, cnt2, it + 1)

    _, _, _, vals, idx, cnt, _ = lax.while_loop(cond, body, state)
    return vals, idx, cnt[:, 0]


def _stage_pass(v, x, s, d, n_arr):
    if d >= 128:
        axis, shift = 0, d // 128
    else:
        axis, shift = 1, d
    is_a = (n_arr & d) == 0
    asc = (n_arr & (1 << s)) == 0
    pv = jnp.where(is_a, jnp.roll(v, -shift, axis=axis),
                   jnp.roll(v, shift, axis=axis))
    px = jnp.where(is_a, jnp.roll(x, -shift, axis=axis),
                   jnp.roll(x, shift, axis=axis))
    a_v = jnp.where(is_a, v, pv)
    b_v = jnp.where(is_a, pv, v)
    a_x = jnp.where(is_a, x, px)
    b_x = jnp.where(is_a, px, x)
    af = (a_v > b_v) | ((a_v == b_v) & (a_x < b_x))
    fv = jnp.where(af, a_v, b_v)
    fx = jnp.where(af, a_x, b_x)
    sv = jnp.where(af, b_v, a_v)
    sx = jnp.where(af, b_x, a_x)
    keep_first = is_a == asc
    return jnp.where(keep_first, fv, sv), jnp.where(keep_first, fx, sx)


def _bitonic_kernel(vin_ref, iin_ref, vout_ref, iout_ref, NLOG=0):
    v = vin_ref[...]
    x = iin_ref[...]
    R = v.shape[0]
    ri = lax.broadcasted_iota(jnp.int32, (R, 128), 0)
    ci = lax.broadcasted_iota(jnp.int32, (R, 128), 1)
    n_arr = ri * 128 + ci
    for s in range(1, NLOG + 1):
        for dp in range(s - 1, -1, -1):
            v, x = _stage_pass(v, x, s, 1 << dp, n_arr)
    vout_ref[...] = v
    iout_ref[...] = x


def _bitonic_sort(vals, idx):
    N = vals.shape[0]
    R = N // 128
    NLOG = N.bit_length() - 1
    v2 = vals.reshape(R, 128)
    i2 = idx.reshape(R, 128)
    vo, io = pl.pallas_call(
        functools.partial(_bitonic_kernel, NLOG=NLOG),
        in_specs=[pl.BlockSpec((R, 128), lambda: (0, 0)),
                  pl.BlockSpec((R, 128), lambda: (0, 0))],
        out_specs=[pl.BlockSpec((R, 128), lambda: (0, 0)),
                   pl.BlockSpec((R, 128), lambda: (0, 0))],
        out_shape=[jax.ShapeDtypeStruct((R, 128), jnp.float32),
                   jax.ShapeDtypeStruct((R, 128), jnp.int32)],
    )(v2, i2)
    return vo.reshape(N), io.reshape(N)


def _mean_kernel(x_ref, out_ref):
    @pl.when(pl.program_id(0) == 0)
    def _():
        out_ref[...] = jnp.zeros_like(out_ref)

    out_ref[...] += jnp.sum(x_ref[...], axis=0, keepdims=True)


def _col_mean(xf):
    s = pl.pallas_call(
        _mean_kernel,
        grid=(GI,),
        in_specs=[pl.BlockSpec((TOK_BLK, D_IN), lambda i: (i, 0))],
        out_specs=pl.BlockSpec((1, D_IN), lambda i: (0, 0)),
        out_shape=jax.ShapeDtypeStruct((1, D_IN), jnp.float32),
    )(xf)
    return s / N_TOK


def _decode_kernel(ta_ref, wd_ref, xf_ref, xbar_ref, bdec_ref,
                   sae_ref, ptl2_ref, pttv_ref, l2_ref, tv_ref, acc):
    k = pl.program_id(1)

    @pl.when(k == 0)
    def _():
        acc[...] = jnp.zeros_like(acc)

    acc[...] += jnp.dot(ta_ref[...], wd_ref[...],
                        preferred_element_type=jnp.float32)

    @pl.when(k == GK - 1)
    def _():
        i = pl.program_id(0)
        sae = acc[...] + bdec_ref[...]
        sae_ref[...] = sae
        xfb = xf_ref[...]
        e = sae - xfb
        ptl2 = jnp.sum(e * e, axis=1)
        d = xfb - xbar_ref[...]
        pttv = jnp.sum(d * d, axis=1)
        ptl2_ref[...] = ptl2.reshape(1, 1, TOK_BLK)
        pttv_ref[...] = pttv.reshape(1, 1, TOK_BLK)

        @pl.when(i == 0)
        def _():
            l2_ref[...] = jnp.zeros_like(l2_ref)
            tv_ref[...] = jnp.zeros_like(tv_ref)

        l2_ref[...] += jnp.sum(ptl2).reshape(1, 1)
        tv_ref[...] += jnp.sum(pttv).reshape(1, 1)


def _decode_stats(top_acts, W_dec, xf, xbar, b_dec):
    return pl.pallas_call(
        _decode_kernel,
        grid=(GI, GK),
        in_specs=[
            pl.BlockSpec((TOK_BLK, LAT_BLK), lambda i, k: (i, k)),
            pl.BlockSpec((LAT_BLK, D_IN), lambda i, k: (k, 0)),
            pl.BlockSpec((TOK_BLK, D_IN), lambda i, k: (i, 0)),
            pl.BlockSpec((1, D_IN), lambda i, k: (0, 0)),
            pl.BlockSpec((1, D_IN), lambda i, k: (0, 0)),
        ],
        out_specs=[
            pl.BlockSpec((TOK_BLK, D_IN), lambda i, k: (i, 0)),
            pl.BlockSpec((1, 1, TOK_BLK), lambda i, k: (i, 0, 0)),
            pl.BlockSpec((1, 1, TOK_BLK), lambda i, k: (i, 0, 0)),
            pl.BlockSpec((1, 1), lambda i, k: (0, 0)),
            pl.BlockSpec((1, 1), lambda i, k: (0, 0)),
        ],
        out_shape=[
            jax.ShapeDtypeStruct((N_TOK, D_IN), jnp.float32),
            jax.ShapeDtypeStruct((GI, 1, TOK_BLK), jnp.float32),
            jax.ShapeDtypeStruct((GI, 1, TOK_BLK), jnp.float32),
            jax.ShapeDtypeStruct((1, 1), jnp.float32),
            jax.ShapeDtypeStruct((1, 1), jnp.float32),
        ],
        scratch_shapes=[pltpu.VMEM((TOK_BLK, D_IN), jnp.float32)],
    )(top_acts, W_dec, xf, xbar, b_dec)


def kernel(x, W_enc, b_enc, W_dec, b_dec):
    B, S, E = x.shape
    xf = x.reshape(B * S, E)

    pre_acts = jax.nn.relu((xf - b_dec) @ W_enc.T + b_enc)

    flat3d, stats = _flatten(pre_acts)
    cand_vals, cand_idx, cand_cnt = _select_candidates(flat3d, stats)

    slot = jnp.arange(CAP_T, dtype=jnp.int32)[None, :]
    valid = slot < cand_cnt[:, None]
    v_flat = jnp.where(valid, cand_vals, -1.0).reshape(-1)
    i_flat = jnp.where(valid, cand_idx, jnp.int32(2**31 - 1)).reshape(-1)

    sv, si = _bitonic_sort(v_flat, i_flat)
    top_vals = sv[:TOTAL_K]
    top_idx = si[:TOTAL_K]

    l0_loss = jnp.sum(stats.reshape(-1, 8)[:, 0]) / N_TOK

    flat_dense = jnp.zeros((NL,), jnp.float32).at[top_idx].set(top_vals)
    top_acts = flat_dense.reshape(N_TOK, NUM_LATENTS)
    top_indices = (top_idx % NUM_LATENTS).reshape(N_TOK, K_TOP)

    xbar = _col_mean(xf)
    sae_out, ptl2_3d, pttv_3d, l2s, tvs = _decode_stats(
        top_acts, W_dec, xf, xbar, b_dec.reshape(1, D_IN))

    per_token_l2_loss = ptl2_3d.reshape(N_TOK)
    per_token_total_variance = pttv_3d.reshape(N_TOK)
    l2_loss = l2s.reshape(())
    total_variance = tvs.reshape(())
    auxk_loss = jnp.float32(0.0)
    fvu = l2_loss / total_variance
    explained_variance = 1.0 - per_token_l2_loss / per_token_total_variance
    return (sae_out, top_acts, top_indices, fvu, l0_loss, l2_loss,
            auxk_loss, explained_variance)
